# baseline (speedup 1.0000x reference)
import functools

import jax
import jax.numpy as jnp
from jax import lax
from jax.experimental import pallas as pl
from jax.experimental.pallas import tpu as pltpu
from jax.experimental.pallas import tpu_sc as plsc

VOCAB = 100000
EMBED = 128
BATCH = 1024

V_TILE = 2000
GRID_V = VOCAB // V_TILE
NBUF = 4


@functools.lru_cache(maxsize=None)
def _make_sc_gather():
    info = plsc.get_sparse_core_info()
    nw = info.num_cores * info.num_subcores
    b_per_w = BATCH // nw
    mesh = plsc.VectorSubcoreMesh(core_axis_name="c", subcore_axis_name="s")

    @functools.partial(
        pl.kernel,
        mesh=mesh,
        out_type=jax.ShapeDtypeStruct((BATCH, EMBED), jnp.float32),
        scratch_types=[
            pltpu.VMEM((b_per_w,), jnp.int32),
            pltpu.VMEM((b_per_w, EMBED), jnp.float32),
            pltpu.SemaphoreType.DMA,
        ],
    )
    def gather(table_hbm, idx_hbm, out_hbm, idx_v, rows_v, sem):
        wid = lax.axis_index("s") * info.num_cores + lax.axis_index("c")
        base = wid * b_per_w
        pltpu.sync_copy(idx_hbm.at[pl.ds(base, b_per_w)], idx_v)
        pltpu.async_copy(table_hbm.at[idx_v], rows_v, sem).wait()
        pltpu.sync_copy(rows_v, out_hbm.at[pl.ds(base, b_per_w)])

    return gather


def _matmul_body(w_ref, x_ref, b_ref, o_hbm, acc, sems):
    i = pl.program_id(0)
    n = pl.num_programs(0)
    slot = lax.rem(i, NBUF)

    def dcopy(s, row):
        return pltpu.make_async_copy(
            acc.at[s], o_hbm.at[pl.ds(row, V_TILE)], sems.at[s])

    @pl.when(i >= NBUF)
    def _():
        dcopy(slot, 0).wait()

    val = lax.dot_general(
        w_ref[...], x_ref[...],
        (((1,), (1,)), ((), ())),
        preferred_element_type=jnp.float32,
    ) + lax.dot_general(
        b_ref[0], jnp.ones((1, BATCH), jnp.float32),
        (((0,), (0,)), ((), ())),
        preferred_element_type=jnp.float32,
    )
    acc[pl.ds(slot, 1)] = val[None]

    dcopy(slot, i * V_TILE).start()

    @pl.when(i == n - 1)
    def _():
        for s in range(NBUF):
            dcopy(s, 0).wait()


def _tc_logits_t(x, fc_w, fc_b2d):
    return pl.pallas_call(
        _matmul_body,
        grid=(GRID_V,),
        in_specs=[
            pl.BlockSpec((V_TILE, EMBED), lambda i: (i, 0)),
            pl.BlockSpec((BATCH, EMBED), lambda i: (0, 0)),
            pl.BlockSpec((1, 1, V_TILE), lambda i: (i, 0, 0)),
        ],
        out_specs=pl.BlockSpec(memory_space=pl.ANY),
        out_shape=jax.ShapeDtypeStruct((VOCAB, BATCH), jnp.float32),
        scratch_shapes=[
            pltpu.VMEM((NBUF, V_TILE, BATCH), jnp.float32),
            pltpu.SemaphoreType.DMA((NBUF,)),
        ],
        compiler_params=pltpu.CompilerParams(
            dimension_semantics=("arbitrary",),
        ),
    )(fc_w, x, fc_b2d)


def kernel(target_word, emb_table, fc_w, fc_b):
    x = _make_sc_gather()(emb_table, target_word.astype(jnp.int32))
    out_t = _tc_logits_t(x, fc_w, fc_b.reshape(GRID_V, 1, V_TILE))
    return out_t.T


# speedup vs baseline: 1.0088x; 1.0088x over previous
import functools

import jax
import jax.numpy as jnp
from jax import lax
from jax.experimental import pallas as pl
from jax.experimental.pallas import tpu as pltpu
from jax.experimental.pallas import tpu_sc as plsc

VOCAB = 100000
EMBED = 128
BATCH = 1024

V_TILE = 4000
GRID_V = VOCAB // V_TILE
NBUF = 3


@functools.lru_cache(maxsize=None)
def _make_sc_gather():
    info = plsc.get_sparse_core_info()
    nw = info.num_cores * info.num_subcores
    b_per_w = BATCH // nw
    mesh = plsc.VectorSubcoreMesh(core_axis_name="c", subcore_axis_name="s")

    @functools.partial(
        pl.kernel,
        mesh=mesh,
        out_type=jax.ShapeDtypeStruct((BATCH, EMBED), jnp.float32),
        scratch_types=[
            pltpu.VMEM((b_per_w,), jnp.int32),
            pltpu.VMEM((b_per_w, EMBED), jnp.float32),
            pltpu.SemaphoreType.DMA,
        ],
    )
    def gather(table_hbm, idx_hbm, out_hbm, idx_v, rows_v, sem):
        wid = lax.axis_index("s") * info.num_cores + lax.axis_index("c")
        base = wid * b_per_w
        pltpu.sync_copy(idx_hbm.at[pl.ds(base, b_per_w)], idx_v)
        pltpu.async_copy(table_hbm.at[idx_v], rows_v, sem).wait()
        pltpu.sync_copy(rows_v, out_hbm.at[pl.ds(base, b_per_w)])

    return gather


def _matmul_body(w_ref, x_ref, b_ref, o_hbm, acc, sems):
    i = pl.program_id(0)
    n = pl.num_programs(0)
    slot = lax.rem(i, NBUF)

    def dcopy(s, row):
        return pltpu.make_async_copy(
            acc.at[s], o_hbm.at[pl.ds(row, V_TILE)], sems.at[s])

    @pl.when(i >= NBUF)
    def _():
        dcopy(slot, 0).wait()

    val = lax.dot_general(
        w_ref[...], x_ref[...],
        (((1,), (1,)), ((), ())),
        preferred_element_type=jnp.float32,
    ) + lax.dot_general(
        b_ref[0], jnp.ones((1, BATCH), jnp.float32),
        (((0,), (0,)), ((), ())),
        preferred_element_type=jnp.float32,
    )
    acc[pl.ds(slot, 1)] = val[None]

    dcopy(slot, i * V_TILE).start()

    @pl.when(i == n - 1)
    def _():
        for s in range(NBUF):
            dcopy(s, 0).wait()


def _tc_logits_t(x, fc_w, fc_b2d):
    return pl.pallas_call(
        _matmul_body,
        grid=(GRID_V,),
        in_specs=[
            pl.BlockSpec((V_TILE, EMBED), lambda i: (i, 0)),
            pl.BlockSpec((BATCH, EMBED), lambda i: (0, 0)),
            pl.BlockSpec((1, 1, V_TILE), lambda i: (i, 0, 0)),
        ],
        out_specs=pl.BlockSpec(memory_space=pl.ANY),
        out_shape=jax.ShapeDtypeStruct((VOCAB, BATCH), jnp.float32),
        scratch_shapes=[
            pltpu.VMEM((NBUF, V_TILE, BATCH), jnp.float32),
            pltpu.SemaphoreType.DMA((NBUF,)),
        ],
        compiler_params=pltpu.CompilerParams(
            dimension_semantics=("arbitrary",),
        ),
    )(fc_w, x, fc_b2d)


def kernel(target_word, emb_table, fc_w, fc_b):
    x = _make_sc_gather()(emb_table, target_word.astype(jnp.int32))
    out_t = _tc_logits_t(x, fc_w, fc_b.reshape(GRID_V, 1, V_TILE))
    return out_t.T


# speedup vs baseline: 1.0127x; 1.0039x over previous
import functools

import jax
import jax.numpy as jnp
from jax import lax
from jax.experimental import pallas as pl
from jax.experimental.pallas import tpu as pltpu
from jax.experimental.pallas import tpu_sc as plsc

VOCAB = 100000
EMBED = 128
BATCH = 1024

V_TILE = 5000
GRID_V = VOCAB // V_TILE
NBUF = 2


@functools.lru_cache(maxsize=None)
def _make_sc_gather():
    info = plsc.get_sparse_core_info()
    nw = info.num_cores * info.num_subcores
    b_per_w = BATCH // nw
    mesh = plsc.VectorSubcoreMesh(core_axis_name="c", subcore_axis_name="s")

    @functools.partial(
        pl.kernel,
        mesh=mesh,
        out_type=jax.ShapeDtypeStruct((BATCH, EMBED), jnp.float32),
        scratch_types=[
            pltpu.VMEM((b_per_w,), jnp.int32),
            pltpu.VMEM((b_per_w, EMBED), jnp.float32),
            pltpu.SemaphoreType.DMA,
        ],
    )
    def gather(table_hbm, idx_hbm, out_hbm, idx_v, rows_v, sem):
        wid = lax.axis_index("s") * info.num_cores + lax.axis_index("c")
        base = wid * b_per_w
        pltpu.sync_copy(idx_hbm.at[pl.ds(base, b_per_w)], idx_v)
        pltpu.async_copy(table_hbm.at[idx_v], rows_v, sem).wait()
        pltpu.sync_copy(rows_v, out_hbm.at[pl.ds(base, b_per_w)])

    return gather


def _matmul_body(w_ref, x_ref, b_ref, o_hbm, acc, sems):
    i = pl.program_id(0)
    n = pl.num_programs(0)
    slot = lax.rem(i, NBUF)

    def dcopy(s, row):
        return pltpu.make_async_copy(
            acc.at[s], o_hbm.at[pl.ds(row, V_TILE)], sems.at[s])

    @pl.when(i >= NBUF)
    def _():
        dcopy(slot, 0).wait()

    val = lax.dot_general(
        w_ref[...], x_ref[...],
        (((1,), (1,)), ((), ())),
        preferred_element_type=jnp.float32,
    ) + lax.dot_general(
        b_ref[0], jnp.ones((1, BATCH), jnp.float32),
        (((0,), (0,)), ((), ())),
        preferred_element_type=jnp.float32,
    )
    acc[pl.ds(slot, 1)] = val[None]

    dcopy(slot, i * V_TILE).start()

    @pl.when(i == n - 1)
    def _():
        for s in range(NBUF):
            dcopy(s, 0).wait()


def _tc_logits_t(x, fc_w, fc_b2d):
    return pl.pallas_call(
        _matmul_body,
        grid=(GRID_V,),
        in_specs=[
            pl.BlockSpec((V_TILE, EMBED), lambda i: (i, 0)),
            pl.BlockSpec((BATCH, EMBED), lambda i: (0, 0)),
            pl.BlockSpec((1, 1, V_TILE), lambda i: (i, 0, 0)),
        ],
        out_specs=pl.BlockSpec(memory_space=pl.ANY),
        out_shape=jax.ShapeDtypeStruct((VOCAB, BATCH), jnp.float32),
        scratch_shapes=[
            pltpu.VMEM((NBUF, V_TILE, BATCH), jnp.float32),
            pltpu.SemaphoreType.DMA((NBUF,)),
        ],
        compiler_params=pltpu.CompilerParams(
            dimension_semantics=("arbitrary",),
        ),
    )(fc_w, x, fc_b2d)


def kernel(target_word, emb_table, fc_w, fc_b):
    x = _make_sc_gather()(emb_table, target_word.astype(jnp.int32))
    out_t = _tc_logits_t(x, fc_w, fc_b.reshape(GRID_V, 1, V_TILE))
    return out_t.T
